# baseline (speedup 1.0000x reference)
import functools

import jax
import jax.numpy as jnp
from jax import lax
from jax.experimental import pallas as pl
from jax.experimental.pallas import tpu as pltpu
from jax.experimental.pallas import tpu_sc as plsc

B = 4096
L = 200
H = 64
HP = 128
N = B * L
CHUNK = 128
LANES = 16
QR = H // LANES


@functools.lru_cache(maxsize=None)
def _build(nc: int, ns: int):
    nw = nc * ns
    per_w = N // nw
    n_chunks = per_w // CHUNK
    assert per_w % CHUNK == 0 and n_chunks % 2 == 0 and per_w % L == 0

    mesh = plsc.VectorSubcoreMesh(
        core_axis_name="c", subcore_axis_name="s",
        num_cores=nc, num_subcores=ns,
    )

    @functools.partial(
        pl.kernel,
        out_type=jax.ShapeDtypeStruct((N, HP), jnp.float32),
        mesh=mesh,
        compiler_params=pltpu.CompilerParams(use_tc_tiling_on_sc=False),
        scratch_types=[
            pltpu.VMEM((n_chunks, CHUNK), jnp.int32),
            pltpu.VMEM((2 * L, H), jnp.float32),
            pltpu.VMEM((CHUNK, H), jnp.float32),
            pltpu.VMEM((CHUNK, H), jnp.float32),
            pltpu.VMEM((CHUNK, HP), jnp.float32),
            pltpu.VMEM((CHUNK, HP), jnp.float32),
            pltpu.SemaphoreType.DMA,
            pltpu.SemaphoreType.DMA,
            pltpu.SemaphoreType.DMA,
            pltpu.SemaphoreType.DMA,
        ],
    )
    def run(idx_hbm, table_hbm, pos2_hbm, out_hbm,
            idx_v, pos_v, gbuf0, gbuf1, obuf0, obuf1, g0, g1, s0, s1):
        wid = lax.axis_index("s") * nc + lax.axis_index("c")
        row0 = wid * per_w

        pltpu.sync_copy(idx_hbm.at[pl.ds(wid * n_chunks, n_chunks)], idx_v)
        pltpu.sync_copy(pos2_hbm, pos_v)

        def start_gather(c, buf, sem):
            pltpu.async_copy(table_hbm.at[idx_v.at[c]], buf, sem)

        def wait_gather(c, buf, sem):
            pltpu.make_async_copy(table_hbm.at[idx_v.at[c]], buf, sem).wait()

        def start_scatter(c, buf, sem):
            pltpu.async_copy(buf, out_hbm.at[pl.ds(row0 + c * CHUNK, CHUNK)], sem)

        def wait_scatter(c, buf, sem):
            pltpu.make_async_copy(
                buf, out_hbm.at[pl.ds(row0 + c * CHUNK, CHUNK)], sem).wait()

        def add_pos(c, gbuf, obuf):
            ph = lax.rem(c * CHUNK, L)

            @plsc.parallel_loop(0, CHUNK, step=1, unroll=8)
            def _(r):
                for q in range(QR):
                    sl = pl.ds(q * LANES, LANES)
                    obuf[r, sl] = gbuf[r, sl] + pos_v[ph + r, sl]

        start_gather(0, gbuf0, g0)

        def cbody(cc, _):
            a = 2 * cc
            b = a + 1

            start_gather(b, gbuf1, g1)
            wait_gather(a, gbuf0, g0)

            @pl.when(cc > 0)
            def _():
                wait_scatter(a - 2, obuf0, s0)
            add_pos(a, gbuf0, obuf0)
            start_scatter(a, obuf0, s0)

            @pl.when(cc < n_chunks // 2 - 1)
            def _():
                start_gather(a + 2, gbuf0, g0)
            wait_gather(b, gbuf1, g1)

            @pl.when(cc > 0)
            def _():
                wait_scatter(b - 2, obuf1, s1)
            add_pos(b, gbuf1, obuf1)
            start_scatter(b, obuf1, s1)
            return 0

        lax.fori_loop(0, n_chunks // 2, cbody, 0)
        wait_scatter(n_chunks - 2, obuf0, s0)
        wait_scatter(n_chunks - 1, obuf1, s1)

    return run


def kernel(input_ids, embed_table, pos_embed):
    info = plsc.get_sparse_core_info()
    run = _build(info.num_cores, info.num_subcores)
    idx = jnp.reshape(input_ids.astype(jnp.int32), (N // CHUNK, CHUNK))
    pos2 = jnp.concatenate([pos_embed, pos_embed], axis=0)
    out = run(idx, embed_table, pos2)
    return jnp.reshape(out[:, :H], (B, L, H))


# speedup vs baseline: 1.0472x; 1.0472x over previous
import functools

import jax
import jax.numpy as jnp
from jax import lax
from jax.experimental import pallas as pl
from jax.experimental.pallas import tpu as pltpu
from jax.experimental.pallas import tpu_sc as plsc

B = 4096
L = 200
H = 64
HP = 128
N = B * L
CHUNK = 128
LANES = 16
QR = H // LANES


@functools.lru_cache(maxsize=None)
def _build(nc: int, ns: int):
    nw = nc * ns
    per_w = N // nw
    n_chunks = per_w // CHUNK
    assert per_w % CHUNK == 0 and n_chunks % 2 == 0 and per_w % L == 0

    mesh = plsc.VectorSubcoreMesh(
        core_axis_name="c", subcore_axis_name="s",
        num_cores=nc, num_subcores=ns,
    )

    @functools.partial(
        pl.kernel,
        out_type=jax.ShapeDtypeStruct((N, HP), jnp.float32),
        mesh=mesh,
        compiler_params=pltpu.CompilerParams(use_tc_tiling_on_sc=False),
        scratch_types=[
            pltpu.VMEM((n_chunks, CHUNK), jnp.int32),
            pltpu.VMEM((2 * L, H), jnp.float32),
            pltpu.VMEM((CHUNK, H), jnp.float32),
            pltpu.VMEM((CHUNK, H), jnp.float32),
            pltpu.VMEM((CHUNK, H), jnp.float32),
            pltpu.VMEM((CHUNK, H), jnp.float32),
            pltpu.SemaphoreType.DMA,
            pltpu.SemaphoreType.DMA,
            pltpu.SemaphoreType.DMA,
            pltpu.SemaphoreType.DMA,
        ],
    )
    def run(idx_hbm, table_hbm, pos2_hbm, out_hbm,
            idx_v, pos_v, gbuf0, gbuf1, obuf0, obuf1, g0, g1, s0, s1):
        wid = lax.axis_index("s") * nc + lax.axis_index("c")
        row0 = wid * per_w

        pltpu.sync_copy(idx_hbm.at[pl.ds(wid * n_chunks, n_chunks)], idx_v)
        pltpu.sync_copy(pos2_hbm, pos_v)

        def start_gather(c, buf, sem):
            pltpu.async_copy(table_hbm.at[idx_v.at[c]], buf, sem)

        def wait_gather(c, buf, sem):
            pltpu.make_async_copy(table_hbm.at[idx_v.at[c]], buf, sem).wait()

        def start_scatter(c, buf, sem):
            pltpu.async_copy(
                buf, out_hbm.at[pl.ds(row0 + c * CHUNK, CHUNK), pl.ds(0, H)],
                sem)

        def wait_scatter(c, buf, sem):
            pltpu.make_async_copy(
                buf, out_hbm.at[pl.ds(row0 + c * CHUNK, CHUNK), pl.ds(0, H)],
                sem).wait()

        def add_pos(c, gbuf, obuf):
            ph = lax.rem(c * CHUNK, L)

            @plsc.parallel_loop(0, CHUNK, step=1, unroll=8)
            def _(r):
                for q in range(QR):
                    sl = pl.ds(q * LANES, LANES)
                    obuf[r, sl] = gbuf[r, sl] + pos_v[ph + r, sl]

        start_gather(0, gbuf0, g0)

        def cbody(cc, _):
            a = 2 * cc
            b = a + 1

            start_gather(b, gbuf1, g1)
            wait_gather(a, gbuf0, g0)

            @pl.when(cc > 0)
            def _():
                wait_scatter(a - 2, obuf0, s0)
            add_pos(a, gbuf0, obuf0)
            start_scatter(a, obuf0, s0)

            @pl.when(cc < n_chunks // 2 - 1)
            def _():
                start_gather(a + 2, gbuf0, g0)
            wait_gather(b, gbuf1, g1)

            @pl.when(cc > 0)
            def _():
                wait_scatter(b - 2, obuf1, s1)
            add_pos(b, gbuf1, obuf1)
            start_scatter(b, obuf1, s1)
            return 0

        lax.fori_loop(0, n_chunks // 2, cbody, 0)
        wait_scatter(n_chunks - 2, obuf0, s0)
        wait_scatter(n_chunks - 1, obuf1, s1)

    return run


def kernel(input_ids, embed_table, pos_embed):
    info = plsc.get_sparse_core_info()
    run = _build(info.num_cores, info.num_subcores)
    idx = jnp.reshape(input_ids.astype(jnp.int32), (N // CHUNK, CHUNK))
    pos2 = jnp.concatenate([pos_embed, pos_embed], axis=0)
    out = run(idx, embed_table, pos2)
    return jnp.reshape(out[:, :H], (B, L, H))


# speedup vs baseline: 1.0917x; 1.0425x over previous
import functools

import jax
import jax.numpy as jnp
from jax import lax
from jax.experimental import pallas as pl
from jax.experimental.pallas import tpu as pltpu
from jax.experimental.pallas import tpu_sc as plsc

B = 4096
L = 200
H = 64
HP = 128
N = B * L
CHUNK = 128
LANES = 16
QR = H // LANES


@functools.lru_cache(maxsize=None)
def _build(nc: int, ns: int):
    nw = nc * ns
    per_w = N // nw
    n_chunks = per_w // CHUNK
    assert per_w % CHUNK == 0 and n_chunks % 2 == 0 and per_w % L == 0

    mesh = plsc.VectorSubcoreMesh(
        core_axis_name="c", subcore_axis_name="s",
        num_cores=nc, num_subcores=ns,
    )

    @functools.partial(
        pl.kernel,
        out_type=jax.ShapeDtypeStruct((N, HP), jnp.float32),
        mesh=mesh,
        compiler_params=pltpu.CompilerParams(use_tc_tiling_on_sc=False),
        scratch_types=[
            pltpu.VMEM((n_chunks, CHUNK), jnp.int32),
            pltpu.VMEM((2 * L, H), jnp.float32),
            [pltpu.VMEM((CHUNK, H), jnp.float32)] * 4,
            [pltpu.VMEM((CHUNK, H), jnp.float32)] * 4,
            [pltpu.SemaphoreType.DMA] * 4,
            [pltpu.SemaphoreType.DMA] * 4,
        ],
    )
    def run(idx_hbm, table_hbm, pos2_hbm, out_hbm,
            idx_v, pos_v, gbufs, obufs, gsems, ssems):
        wid = lax.axis_index("s") * nc + lax.axis_index("c")
        row0 = wid * per_w

        pltpu.sync_copy(idx_hbm.at[pl.ds(wid * n_chunks, n_chunks)], idx_v)
        pltpu.sync_copy(pos2_hbm, pos_v)

        def start_gather(c, buf, sem):
            pltpu.async_copy(table_hbm.at[idx_v.at[c]], buf, sem)

        def wait_gather(c, buf, sem):
            pltpu.make_async_copy(table_hbm.at[idx_v.at[c]], buf, sem).wait()

        def start_scatter(c, buf, sem):
            pltpu.async_copy(
                buf, out_hbm.at[pl.ds(row0 + c * CHUNK, CHUNK), pl.ds(0, H)],
                sem)

        def wait_scatter(c, buf, sem):
            pltpu.make_async_copy(
                buf, out_hbm.at[pl.ds(row0 + c * CHUNK, CHUNK), pl.ds(0, H)],
                sem).wait()

        def add_pos(c, gbuf, obuf):
            ph = lax.rem(c * CHUNK, L)

            @plsc.parallel_loop(0, CHUNK, step=1, unroll=8)
            def _(r):
                for q in range(QR):
                    sl = pl.ds(q * LANES, LANES)
                    obuf[r, sl] = gbuf[r, sl] + pos_v[ph + r, sl]

        nbuf = 4
        for k in range(nbuf):
            start_gather(k, gbufs[k], gsems[k])

        def cbody(cc, _):
            for k in range(nbuf):
                c = nbuf * cc + k
                wait_gather(c, gbufs[k], gsems[k])

                @pl.when(cc > 0)
                def _(k=k, c=c):
                    wait_scatter(c - nbuf, obufs[k], ssems[k])
                add_pos(c, gbufs[k], obufs[k])
                start_scatter(c, obufs[k], ssems[k])

                @pl.when(cc < n_chunks // nbuf - 1)
                def _(k=k, c=c):
                    start_gather(c + nbuf, gbufs[k], gsems[k])
            return 0

        lax.fori_loop(0, n_chunks // nbuf, cbody, 0)
        for k in range(nbuf):
            wait_scatter(n_chunks - nbuf + k, obufs[k], ssems[k])

    return run


def kernel(input_ids, embed_table, pos_embed):
    info = plsc.get_sparse_core_info()
    run = _build(info.num_cores, info.num_subcores)
    idx = jnp.reshape(input_ids.astype(jnp.int32), (N // CHUNK, CHUNK))
    pos2 = jnp.concatenate([pos_embed, pos_embed], axis=0)
    out = run(idx, embed_table, pos2)
    return jnp.reshape(out[:, :H], (B, L, H))


# speedup vs baseline: 1.0925x; 1.0007x over previous
import functools

import jax
import jax.numpy as jnp
from jax import lax
from jax.experimental import pallas as pl
from jax.experimental.pallas import tpu as pltpu
from jax.experimental.pallas import tpu_sc as plsc

B = 4096
L = 200
H = 64
HP = 128
N = B * L
CHUNK = 128
LANES = 16
QR = H // LANES
NBUF = 5
POSR = L + CHUNK - 8


@functools.lru_cache(maxsize=None)
def _build(nc: int, ns: int):
    nw = nc * ns
    per_w = N // nw
    n_chunks = per_w // CHUNK
    assert per_w % CHUNK == 0 and n_chunks % NBUF == 0 and per_w % L == 0

    mesh = plsc.VectorSubcoreMesh(
        core_axis_name="c", subcore_axis_name="s",
        num_cores=nc, num_subcores=ns,
    )

    @functools.partial(
        pl.kernel,
        out_type=jax.ShapeDtypeStruct((N, HP), jnp.float32),
        mesh=mesh,
        compiler_params=pltpu.CompilerParams(use_tc_tiling_on_sc=False),
        scratch_types=[
            pltpu.VMEM((n_chunks, CHUNK), jnp.int32),
            pltpu.VMEM((POSR, H), jnp.float32),
            [pltpu.VMEM((CHUNK, H), jnp.float32)] * NBUF,
            [pltpu.VMEM((CHUNK, H), jnp.float32)] * NBUF,
            [pltpu.SemaphoreType.DMA] * NBUF,
            [pltpu.SemaphoreType.DMA] * NBUF,
        ],
    )
    def run(idx_hbm, table_hbm, pos2_hbm, out_hbm,
            idx_v, pos_v, gbufs, obufs, gsems, ssems):
        wid = lax.axis_index("s") * nc + lax.axis_index("c")
        row0 = wid * per_w

        pltpu.sync_copy(idx_hbm.at[pl.ds(wid * n_chunks, n_chunks)], idx_v)
        pltpu.sync_copy(pos2_hbm, pos_v)

        def start_gather(c, buf, sem):
            pltpu.async_copy(table_hbm.at[idx_v.at[c]], buf, sem)

        def wait_gather(c, buf, sem):
            pltpu.make_async_copy(table_hbm.at[idx_v.at[c]], buf, sem).wait()

        def start_scatter(c, buf, sem):
            pltpu.async_copy(
                buf, out_hbm.at[pl.ds(row0 + c * CHUNK, CHUNK), pl.ds(0, H)],
                sem)

        def wait_scatter(c, buf, sem):
            pltpu.make_async_copy(
                buf, out_hbm.at[pl.ds(row0 + c * CHUNK, CHUNK), pl.ds(0, H)],
                sem).wait()

        def add_pos(c, gbuf, obuf):
            ph = lax.rem(c * CHUNK, L)

            @plsc.parallel_loop(0, CHUNK, step=1, unroll=8)
            def _(r):
                for q in range(QR):
                    sl = pl.ds(q * LANES, LANES)
                    obuf[r, sl] = gbuf[r, sl] + pos_v[ph + r, sl]

        nbuf = NBUF
        for k in range(nbuf):
            start_gather(k, gbufs[k], gsems[k])

        def cbody(cc, _):
            for k in range(nbuf):
                c = nbuf * cc + k
                wait_gather(c, gbufs[k], gsems[k])

                @pl.when(cc > 0)
                def _(k=k, c=c):
                    wait_scatter(c - nbuf, obufs[k], ssems[k])
                add_pos(c, gbufs[k], obufs[k])
                start_scatter(c, obufs[k], ssems[k])

                @pl.when(cc < n_chunks // nbuf - 1)
                def _(k=k, c=c):
                    start_gather(c + nbuf, gbufs[k], gsems[k])
            return 0

        lax.fori_loop(0, n_chunks // nbuf, cbody, 0)
        for k in range(nbuf):
            wait_scatter(n_chunks - nbuf + k, obufs[k], ssems[k])

    return run


def kernel(input_ids, embed_table, pos_embed):
    info = plsc.get_sparse_core_info()
    run = _build(info.num_cores, info.num_subcores)
    idx = jnp.reshape(input_ids.astype(jnp.int32), (N // CHUNK, CHUNK))
    pos2 = jnp.concatenate([pos_embed, pos_embed], axis=0)[:POSR]
    out = run(idx, embed_table, pos2)
    return jnp.reshape(out[:, :H], (B, L, H))
